# Initial kernel scaffold; baseline (speedup 1.0000x reference)
import jax
import jax.numpy as jnp
from jax.experimental import pallas as pl


def kernel(node_feats, edge_index, W0, a0, g0, b0, W1, a1, g1, b1, W2, a2, g2, b2):
    raise NotImplementedError("write your pallas kernel here")


# speedup vs baseline: 64.9943x; 64.9943x over previous
import functools

import jax
import jax.numpy as jnp
from jax import lax
from jax.experimental import pallas as pl
from jax.experimental.pallas import tpu as pltpu
from jax.experimental.pallas import tpu_sc as plsc

_HEADS = 4
_HD = 32
_L = 16
_NC = 2
_NS = 16


def _pre_body(x_ref, w_ref, a_ref, wx2_ref, s_ref):
    wx = jnp.dot(x_ref[...], w_ref[...], preferred_element_type=jnp.float32)
    wx2_ref[0] = wx[:, :64]
    wx2_ref[1] = wx[:, 64:]
    s_ref[...] = jnp.dot(wx, a_ref[...], preferred_element_type=jnp.float32)


def _make_pre(n, r):
    return pl.pallas_call(
        _pre_body,
        grid=(n // r,),
        in_specs=[
            pl.BlockSpec((r, 128), lambda i: (i, 0)),
            pl.BlockSpec((128, 128), lambda i: (0, 0)),
            pl.BlockSpec((128, 8), lambda i: (0, 0)),
        ],
        out_specs=[
            pl.BlockSpec((_NC, r, 64), lambda i: (0, i, 0)),
            pl.BlockSpec((r, 8), lambda i: (i, 0)),
        ],
        out_shape=[
            jax.ShapeDtypeStruct((_NC, n, 64), jnp.float32),
            jax.ShapeDtypeStruct((n, 8), jnp.float32),
        ],
    )


def _post_body(num_ref, den_ref, p_ref, g_ref, b_ref, o_ref):
    num = jnp.concatenate([num_ref[0], num_ref[1]], axis=-1)
    den_exp = jnp.dot(den_ref[...], p_ref[...],
                      preferred_element_type=jnp.float32)
    h = num / (den_exp + 1e-8)
    h = jnp.where(h > 0, h, jnp.exp(h) - 1.0)
    mu = jnp.mean(h, axis=-1, keepdims=True)
    var = jnp.mean((h - mu) ** 2, axis=-1, keepdims=True)
    o_ref[...] = (h - mu) / jnp.sqrt(var + 1e-5) * g_ref[...] + b_ref[...]


def _make_post(n, r):
    return pl.pallas_call(
        _post_body,
        grid=(n // r,),
        in_specs=[
            pl.BlockSpec((_NC, r, 64), lambda i: (0, i, 0)),
            pl.BlockSpec((r, 8), lambda i: (i, 0)),
            pl.BlockSpec((8, 128), lambda i: (0, 0)),
            pl.BlockSpec((128,), lambda i: (0,)),
            pl.BlockSpec((128,), lambda i: (0,)),
        ],
        out_specs=pl.BlockSpec((r, 128), lambda i: (i, 0)),
        out_shape=jax.ShapeDtypeStruct((n, 128), jnp.float32),
    )


def _make_sc(n, e, k):
    epw = e // _NS
    nchunks = epw // k
    rpt = n // _NS
    mesh = plsc.VectorSubcoreMesh(
        core_axis_name="c", subcore_axis_name="s",
        num_cores=_NC, num_subcores=_NS)

    @functools.partial(
        pl.kernel,
        out_type=[
            jax.ShapeDtypeStruct((_NC, n, 64), jnp.float32),
            jax.ShapeDtypeStruct((n, 8), jnp.float32),
        ],
        mesh=mesh,
        scratch_types=[
            pltpu.VMEM((k,), jnp.int32),
            pltpu.VMEM((k,), jnp.int32),
            pltpu.VMEM((k, 64), jnp.float32),
            pltpu.VMEM((k, 8), jnp.float32),
            pltpu.VMEM((k, 8), jnp.float32),
            pltpu.VMEM((k, 8), jnp.float32),
            pltpu.VMEM_SHARED((n, 64), jnp.float32),
            pltpu.VMEM_SHARED((n, 8), jnp.float32),
            pltpu.VMEM_SHARED((n, 8), jnp.float32),
            pltpu.SemaphoreType.DMA,
        ],
        compiler_params=pltpu.CompilerParams(
            needs_layout_passes=False, use_tc_tiling_on_sc=False),
    )
    def sc(wx2_hbm, s_hbm, src_hbm, dst_hbm, z64_hbm, z8_hbm,
           num_hbm, den_hbm,
           srcv, dstv, rows, ssb, sdb, wbuf, num_sh, den_sh, s_sh, sem):
        cid = lax.axis_index("c")
        sid = lax.axis_index("s")
        nb = sid * rpt
        pltpu.sync_copy(z64_hbm, num_sh.at[pl.ds(nb, rpt)])
        pltpu.sync_copy(z8_hbm, den_sh.at[pl.ds(nb, rpt)])
        pltpu.sync_copy(s_hbm.at[pl.ds(nb, rpt)], s_sh.at[pl.ds(nb, rpt)])
        pltpu.sync_copy(z8_hbm.at[pl.ds(0, k)], wbuf)
        plsc.subcore_barrier()

        ebase = sid * epw
        my_wx = wx2_hbm.at[cid]

        def chunk(i, carry):
            cb = ebase + i * k
            pltpu.sync_copy(src_hbm.at[pl.ds(cb, k)], srcv)
            pltpu.sync_copy(dst_hbm.at[pl.ds(cb, k)], dstv)
            gat = pltpu.async_copy(my_wx.at[srcv], rows, sem)
            pltpu.sync_copy(s_sh.at[srcv], ssb)
            pltpu.sync_copy(s_sh.at[dstv], sdb)

            def wgrp(g, c2):
                idx = lax.iota(jnp.int32, _L) + g * _L
                for h in range(_HEADS):
                    hv = jnp.full((_L,), h, jnp.int32)
                    sa = plsc.load_gather(ssb, [idx, hv])
                    sb = plsc.load_gather(sdb, [idx, hv + 4])
                    ee = sa + sb
                    ee = jnp.maximum(ee, 0.2 * ee)
                    plsc.store_scatter(wbuf, [idx, hv], jnp.exp(ee))
                return c2
            lax.fori_loop(0, k // _L, wgrp, 0)
            gat.wait()

            def scale(j, c2):
                jv = jnp.full((_L,), j, jnp.int32)
                for h in range(2):
                    hv = jnp.full((_L,), h, jnp.int32) + 2 * cid
                    w = plsc.load_gather(wbuf, [jv, hv])
                    for half in range(2):
                        c0 = h * _HD + half * _L
                        rows[j, pl.ds(c0, _L)] = rows[j, pl.ds(c0, _L)] * w
                return c2
            lax.fori_loop(0, k, scale, 0)

            pltpu.sync_copy(rows, num_sh.at[dstv], add=True)

            @pl.when(cid == 0)
            def _():
                pltpu.sync_copy(wbuf, den_sh.at[dstv], add=True)
            return carry
        lax.fori_loop(0, nchunks, chunk, 0)

        plsc.subcore_barrier()
        pltpu.sync_copy(num_sh.at[pl.ds(nb, rpt)],
                        num_hbm.at[cid, pl.ds(nb, rpt)])

        @pl.when(cid == 0)
        def _():
            pltpu.sync_copy(den_sh.at[pl.ds(nb, rpt)],
                            den_hbm.at[pl.ds(nb, rpt)])

    return sc


def _build_attn_mat(a):
    a_src = a[:_HD, 0].reshape(_HD, 1)
    a_dst = a[_HD:, 0].reshape(_HD, 1)
    eye = jnp.eye(_HEADS, dtype=jnp.float32)
    return jnp.concatenate(
        [jnp.kron(eye, a_src), jnp.kron(eye, a_dst)], axis=1)


def kernel(node_feats, edge_index, W0, a0, g0, b0, W1, a1, g1, b1,
           W2, a2, g2, b2):
    n, d = node_feats.shape
    e = edge_index.shape[1]
    assert d == 128 and e % _NS == 0
    npad = ((n + _NS * 8 - 1) // (_NS * 8)) * (_NS * 8)

    src = edge_index[0]
    dst = edge_index[1]
    z64 = jnp.zeros((npad // _NS, 64), jnp.float32)
    z8 = jnp.zeros((npad // _NS, 8), jnp.float32)
    p8 = jnp.concatenate(
        [jnp.kron(jnp.eye(_HEADS, dtype=jnp.float32),
                  jnp.ones((1, _HD), jnp.float32)),
         jnp.zeros((4, 128), jnp.float32)], axis=0)

    r = 1024
    pre = _make_pre(npad, r)
    post = _make_post(npad, r)
    sc = _make_sc(npad, e, 200)

    x = jnp.pad(node_feats, ((0, npad - n), (0, 0)))
    for (W, a, g, b) in ((W0, a0, g0, b0), (W1, a1, g1, b1), (W2, a2, g2, b2)):
        wx2, s = pre(x, W, _build_attn_mat(a))
        num, den = sc(wx2, s, src, dst, z64, z8)
        x = post(num, den, p8, g, b)
    return x[:n]


# speedup vs baseline: 72.4178x; 1.1142x over previous
import functools

import jax
import jax.numpy as jnp
from jax import lax
from jax.experimental import pallas as pl
from jax.experimental.pallas import tpu as pltpu
from jax.experimental.pallas import tpu_sc as plsc

_HEADS = 4
_HD = 32
_L = 16
_NC = 2
_NS = 16


def _pre_body(x_ref, w_ref, a_ref, wx2_ref, s_ref):
    wx = jnp.dot(x_ref[...], w_ref[...], preferred_element_type=jnp.float32)
    wx2_ref[0] = wx[:, :64]
    wx2_ref[1] = wx[:, 64:]
    s_ref[...] = jnp.dot(wx, a_ref[...], preferred_element_type=jnp.float32)


def _make_pre(n, r):
    return pl.pallas_call(
        _pre_body,
        grid=(n // r,),
        in_specs=[
            pl.BlockSpec((r, 128), lambda i: (i, 0)),
            pl.BlockSpec((128, 128), lambda i: (0, 0)),
            pl.BlockSpec((128, 8), lambda i: (0, 0)),
        ],
        out_specs=[
            pl.BlockSpec((_NC, r, 64), lambda i: (0, i, 0)),
            pl.BlockSpec((r, 8), lambda i: (i, 0)),
        ],
        out_shape=[
            jax.ShapeDtypeStruct((_NC, n, 64), jnp.float32),
            jax.ShapeDtypeStruct((n, 8), jnp.float32),
        ],
    )


def _post_body(num_ref, den_ref, p_ref, g_ref, b_ref, o_ref):
    num = jnp.concatenate([num_ref[0], num_ref[1]], axis=-1)
    den_exp = jnp.dot(den_ref[...], p_ref[...],
                      preferred_element_type=jnp.float32)
    h = num / (den_exp + 1e-8)
    h = jnp.where(h > 0, h, jnp.exp(h) - 1.0)
    mu = jnp.mean(h, axis=-1, keepdims=True)
    var = jnp.mean((h - mu) ** 2, axis=-1, keepdims=True)
    o_ref[...] = (h - mu) / jnp.sqrt(var + 1e-5) * g_ref[...] + b_ref[...]


def _make_post(n, r):
    return pl.pallas_call(
        _post_body,
        grid=(n // r,),
        in_specs=[
            pl.BlockSpec((_NC, r, 64), lambda i: (0, i, 0)),
            pl.BlockSpec((r, 8), lambda i: (i, 0)),
            pl.BlockSpec((8, 128), lambda i: (0, 0)),
            pl.BlockSpec((128,), lambda i: (0,)),
            pl.BlockSpec((128,), lambda i: (0,)),
        ],
        out_specs=pl.BlockSpec((r, 128), lambda i: (i, 0)),
        out_shape=jax.ShapeDtypeStruct((n, 128), jnp.float32),
    )


def _make_sc(n, e, k):
    assert k % _L == 0
    epw = e // _NS
    assert epw % k == 0
    nchunks = epw // k
    rpt = n // _NS
    mesh = plsc.VectorSubcoreMesh(
        core_axis_name="c", subcore_axis_name="s",
        num_cores=_NC, num_subcores=_NS)

    @functools.partial(
        pl.kernel,
        out_type=[
            jax.ShapeDtypeStruct((_NC, n, 64), jnp.float32),
            jax.ShapeDtypeStruct((n, 8), jnp.float32),
        ],
        mesh=mesh,
        scratch_types=[
            pltpu.VMEM((k,), jnp.int32),
            pltpu.VMEM((k,), jnp.int32),
            pltpu.VMEM((k, 64), jnp.float32),
            pltpu.VMEM((k, 8), jnp.float32),
            pltpu.VMEM((k, 8), jnp.float32),
            pltpu.VMEM((k, 8), jnp.float32),
            pltpu.VMEM_SHARED((n, 64), jnp.float32),
            pltpu.VMEM_SHARED((n, 8), jnp.float32),
            pltpu.VMEM_SHARED((n, 8), jnp.float32),
            pltpu.SemaphoreType.DMA,
        ],
        compiler_params=pltpu.CompilerParams(
            needs_layout_passes=False, use_tc_tiling_on_sc=False),
    )
    def sc(wx2_hbm, s_hbm, src_hbm, dst_hbm, z64_hbm, z8_hbm,
           num_hbm, den_hbm,
           srcv, dstv, rows, ssb, sdb, wbuf, num_sh, den_sh, s_sh, sem):
        cid = lax.axis_index("c")
        sid = lax.axis_index("s")
        nb = sid * rpt
        pltpu.sync_copy(z64_hbm, num_sh.at[pl.ds(nb, rpt)])
        pltpu.sync_copy(z8_hbm, den_sh.at[pl.ds(nb, rpt)])
        pltpu.sync_copy(s_hbm.at[pl.ds(nb, rpt)], s_sh.at[pl.ds(nb, rpt)])
        pltpu.sync_copy(z8_hbm.at[pl.ds(0, k)], wbuf)
        plsc.subcore_barrier()

        ebase = sid * epw
        my_wx = wx2_hbm.at[cid]

        def chunk(i, carry):
            cb = ebase + i * k
            pltpu.sync_copy(src_hbm.at[pl.ds(cb, k)], srcv)
            pltpu.sync_copy(dst_hbm.at[pl.ds(cb, k)], dstv)
            gat = pltpu.async_copy(my_wx.at[srcv], rows, sem)
            pltpu.sync_copy(s_sh.at[srcv], ssb)
            pltpu.sync_copy(s_sh.at[dstv], sdb)

            def wgrp(g, c2):
                idx = lax.iota(jnp.int32, _L) + g * _L
                for h in range(_HEADS):
                    hv = jnp.full((_L,), h, jnp.int32)
                    sa = plsc.load_gather(ssb, [idx, hv])
                    sb = plsc.load_gather(sdb, [idx, hv + 4])
                    ee = sa + sb
                    ee = jnp.maximum(ee, 0.2 * ee)
                    plsc.store_scatter(wbuf, [idx, hv], jnp.exp(ee))
                return c2
            lax.fori_loop(0, k // _L, wgrp, 0)
            gat.wait()

            def scale(j, c2):
                jv = jnp.full((_L,), j, jnp.int32)
                for h in range(2):
                    hv = jnp.full((_L,), h, jnp.int32) + 2 * cid
                    w = plsc.load_gather(wbuf, [jv, hv])
                    for half in range(2):
                        c0 = h * _HD + half * _L
                        rows[j, pl.ds(c0, _L)] = rows[j, pl.ds(c0, _L)] * w
                return c2
            lax.fori_loop(0, k, scale, 0)

            pltpu.sync_copy(rows, num_sh.at[dstv], add=True)

            @pl.when(cid == 0)
            def _():
                pltpu.sync_copy(wbuf, den_sh.at[dstv], add=True)
            return carry
        lax.fori_loop(0, nchunks, chunk, 0)

        plsc.subcore_barrier()
        pltpu.sync_copy(num_sh.at[pl.ds(nb, rpt)],
                        num_hbm.at[cid, pl.ds(nb, rpt)])

        @pl.when(cid == 0)
        def _():
            pltpu.sync_copy(den_sh.at[pl.ds(nb, rpt)],
                            den_hbm.at[pl.ds(nb, rpt)])

    return sc


def _build_attn_mat(a):
    a_src = a[:_HD, 0].reshape(_HD, 1)
    a_dst = a[_HD:, 0].reshape(_HD, 1)
    eye = jnp.eye(_HEADS, dtype=jnp.float32)
    return jnp.concatenate(
        [jnp.kron(eye, a_src), jnp.kron(eye, a_dst)], axis=1)


def kernel(node_feats, edge_index, W0, a0, g0, b0, W1, a1, g1, b1,
           W2, a2, g2, b2):
    n, d = node_feats.shape
    e = edge_index.shape[1]
    assert d == 128 and e % _NS == 0
    r = 1024
    npad = ((n + r - 1) // r) * r

    src = edge_index[0]
    dst = edge_index[1]
    z64 = jnp.zeros((npad // _NS, 64), jnp.float32)
    z8 = jnp.zeros((npad // _NS, 8), jnp.float32)
    p8 = jnp.concatenate(
        [jnp.kron(jnp.eye(_HEADS, dtype=jnp.float32),
                  jnp.ones((1, _HD), jnp.float32)),
         jnp.zeros((4, 128), jnp.float32)], axis=0)

    pre = _make_pre(npad, r)
    post = _make_post(npad, r)
    sc = _make_sc(npad, e, 400)

    x = jnp.pad(node_feats, ((0, npad - n), (0, 0)))
    for (W, a, g, b) in ((W0, a0, g0, b0), (W1, a1, g1, b1), (W2, a2, g2, b2)):
        wx2, s = pre(x, W, _build_attn_mat(a))
        num, den = sc(wx2, s, src, dst, z64, z8)
        x = post(num, den, p8, g, b)
    return x[:n]


# speedup vs baseline: 78.2700x; 1.0808x over previous
import functools

import jax
import jax.numpy as jnp
from jax import lax
from jax.experimental import pallas as pl
from jax.experimental.pallas import tpu as pltpu
from jax.experimental.pallas import tpu_sc as plsc

_HEADS = 4
_HD = 32
_L = 16
_NC = 2
_NS = 16


def _pre_body(x_ref, w_ref, a_ref, wx2_ref, s_ref):
    wx = jnp.dot(x_ref[...], w_ref[...], preferred_element_type=jnp.float32)
    wx2_ref[0] = wx[:, :64]
    wx2_ref[1] = wx[:, 64:]
    s_ref[...] = jnp.dot(wx, a_ref[...], preferred_element_type=jnp.float32)


def _make_pre(n, r):
    return pl.pallas_call(
        _pre_body,
        grid=(n // r,),
        in_specs=[
            pl.BlockSpec((r, 128), lambda i: (i, 0)),
            pl.BlockSpec((128, 128), lambda i: (0, 0)),
            pl.BlockSpec((128, 8), lambda i: (0, 0)),
        ],
        out_specs=[
            pl.BlockSpec((_NC, r, 64), lambda i: (0, i, 0)),
            pl.BlockSpec((r, 8), lambda i: (i, 0)),
        ],
        out_shape=[
            jax.ShapeDtypeStruct((_NC, n, 64), jnp.float32),
            jax.ShapeDtypeStruct((n, 8), jnp.float32),
        ],
    )


def _post_body(num_ref, den_ref, p_ref, g_ref, b_ref, o_ref):
    num = jnp.concatenate([num_ref[0], num_ref[1]], axis=-1)
    den_exp = jnp.dot(den_ref[...], p_ref[...],
                      preferred_element_type=jnp.float32)
    h = num / (den_exp + 1e-8)
    h = jnp.where(h > 0, h, jnp.exp(h) - 1.0)
    mu = jnp.mean(h, axis=-1, keepdims=True)
    var = jnp.mean((h - mu) ** 2, axis=-1, keepdims=True)
    o_ref[...] = (h - mu) / jnp.sqrt(var + 1e-5) * g_ref[...] + b_ref[...]


def _make_post(n, r):
    return pl.pallas_call(
        _post_body,
        grid=(n // r,),
        in_specs=[
            pl.BlockSpec((_NC, r, 64), lambda i: (0, i, 0)),
            pl.BlockSpec((r, 8), lambda i: (i, 0)),
            pl.BlockSpec((8, 128), lambda i: (0, 0)),
            pl.BlockSpec((128,), lambda i: (0,)),
            pl.BlockSpec((128,), lambda i: (0,)),
        ],
        out_specs=pl.BlockSpec((r, 128), lambda i: (i, 0)),
        out_shape=jax.ShapeDtypeStruct((n, 128), jnp.float32),
    )


def _make_sc(n, e, k):
    assert k % _L == 0
    epw = e // _NS
    assert epw % k == 0
    ncl = epw // k
    assert ncl % 2 == 0
    rpt = n // _NS
    mesh = plsc.VectorSubcoreMesh(
        core_axis_name="c", subcore_axis_name="s",
        num_cores=_NC, num_subcores=_NS)

    @functools.partial(
        pl.kernel,
        out_type=[
            jax.ShapeDtypeStruct((_NC, n, 64), jnp.float32),
            jax.ShapeDtypeStruct((n, 8), jnp.float32),
        ],
        mesh=mesh,
        scratch_types=[
            pltpu.VMEM((k,), jnp.int32),
            pltpu.VMEM((k,), jnp.int32),
            pltpu.VMEM((k,), jnp.int32),
            pltpu.VMEM((k,), jnp.int32),
            pltpu.VMEM((k, 64), jnp.float32),
            pltpu.VMEM((k, 64), jnp.float32),
            pltpu.VMEM((k, 8), jnp.float32),
            pltpu.VMEM((k, 8), jnp.float32),
            pltpu.VMEM((k, 8), jnp.float32),
            pltpu.VMEM((k, 8), jnp.float32),
            pltpu.VMEM((k, 8), jnp.float32),
            pltpu.VMEM((k, 8), jnp.float32),
            pltpu.VMEM_SHARED((n, 64), jnp.float32),
            pltpu.VMEM_SHARED((n, 8), jnp.float32),
            pltpu.VMEM_SHARED((n, 8), jnp.float32),
            pltpu.SemaphoreType.DMA,
            pltpu.SemaphoreType.DMA,
            pltpu.SemaphoreType.DMA,
            pltpu.SemaphoreType.DMA,
        ],
        compiler_params=pltpu.CompilerParams(
            needs_layout_passes=False, use_tc_tiling_on_sc=False),
    )
    def sc(wx2_hbm, s_hbm, src_hbm, dst_hbm, z64_hbm, z8_hbm,
           num_hbm, den_hbm,
           srcv_a, srcv_b, dstv_a, dstv_b,
           rows_a, rows_b, ssb_a, ssb_b, sdb_a, sdb_b, wbuf_a, wbuf_b,
           num_sh, den_sh, s_sh,
           isem_a, isem_b, rsem, ssem):
        cid = lax.axis_index("c")
        sid = lax.axis_index("s")
        nb = sid * rpt
        srcv = (srcv_a, srcv_b)
        dstv = (dstv_a, dstv_b)
        rows = (rows_a, rows_b)
        ssb = (ssb_a, ssb_b)
        sdb = (sdb_a, sdb_b)
        wbuf = (wbuf_a, wbuf_b)
        isem = (isem_a, isem_b)
        my_wx = wx2_hbm.at[cid]
        ebase = sid * epw

        pltpu.sync_copy(z64_hbm, num_sh.at[pl.ds(nb, rpt)])
        pltpu.sync_copy(z8_hbm, den_sh.at[pl.ds(nb, rpt)])
        pltpu.sync_copy(s_hbm.at[pl.ds(nb, rpt)], s_sh.at[pl.ds(nb, rpt)])
        pltpu.sync_copy(z8_hbm.at[pl.ds(0, k)], wbuf_a)
        pltpu.sync_copy(z8_hbm.at[pl.ds(0, k)], wbuf_b)
        plsc.subcore_barrier()

        def issue_idx(ci, p):
            cb = ebase + ci * k
            pltpu.async_copy(src_hbm.at[pl.ds(cb, k)], srcv[p], isem[p])
            pltpu.async_copy(dst_hbm.at[pl.ds(cb, k)], dstv[p], isem[p])

        def drain_idx(p):
            pltpu.make_async_copy(src_hbm.at[pl.ds(0, k)], srcv[p],
                                  isem[p]).wait()
            pltpu.make_async_copy(dst_hbm.at[pl.ds(0, k)], dstv[p],
                                  isem[p]).wait()

        def process(j, p, d_rows, d_ssb, d_sdb):
            d_ssb.wait()
            d_sdb.wait()

            def wgrp(g, c2):
                idx = lax.iota(jnp.int32, _L) + g * _L
                for h in range(_HEADS):
                    hv = jnp.full((_L,), h, jnp.int32)
                    sa = plsc.load_gather(ssb[p], [idx, hv])
                    sb = plsc.load_gather(sdb[p], [idx, hv + 4])
                    ee = sa + sb
                    ee = jnp.maximum(ee, 0.2 * ee)
                    plsc.store_scatter(wbuf[p], [idx, hv], jnp.exp(ee))
                return c2
            lax.fori_loop(0, k // _L, wgrp, 0)

            d_rows.wait()
            rp = rows[p]

            def scale(i, c2):
                iv = jnp.full((_L,), i, jnp.int32)
                for h in range(2):
                    hv = jnp.full((_L,), h, jnp.int32) + 2 * cid
                    w = plsc.load_gather(wbuf[p], [iv, hv])
                    for half in range(2):
                        c0 = h * _HD + half * _L
                        rp[i, pl.ds(c0, _L)] = rp[i, pl.ds(c0, _L)] * w
                return c2
            lax.fori_loop(0, k, scale, 0)

            pltpu.sync_copy(rows[p], num_sh.at[dstv[p]], add=True)

            @pl.when(cid == 0)
            def _():
                pltpu.sync_copy(wbuf[p], den_sh.at[dstv[p]], add=True)

            @pl.when(j + 2 < ncl)
            def _():
                issue_idx(j + 2, p)

        issue_idx(0, 0)
        issue_idx(1, 1)

        def pair(t, carry):
            j0 = 2 * t
            drain_idx(0)
            drain_idx(1)
            d_r0 = pltpu.async_copy(my_wx.at[srcv[0]], rows[0], rsem)
            d_s0 = pltpu.async_copy(s_sh.at[srcv[0]], ssb[0], ssem)
            d_d0 = pltpu.async_copy(s_sh.at[dstv[0]], sdb[0], ssem)
            d_r1 = pltpu.async_copy(my_wx.at[srcv[1]], rows[1], rsem)
            d_s1 = pltpu.async_copy(s_sh.at[srcv[1]], ssb[1], ssem)
            d_d1 = pltpu.async_copy(s_sh.at[dstv[1]], sdb[1], ssem)
            process(j0, 0, d_r0, d_s0, d_d0)
            process(j0 + 1, 1, d_r1, d_s1, d_d1)
            return carry
        lax.fori_loop(0, ncl // 2, pair, 0)

        plsc.subcore_barrier()
        pltpu.sync_copy(num_sh.at[pl.ds(nb, rpt)],
                        num_hbm.at[cid, pl.ds(nb, rpt)])

        @pl.when(cid == 0)
        def _():
            pltpu.sync_copy(den_sh.at[pl.ds(nb, rpt)],
                            den_hbm.at[pl.ds(nb, rpt)])

    return sc


def _build_attn_mat(a):
    a_src = a[:_HD, 0].reshape(_HD, 1)
    a_dst = a[_HD:, 0].reshape(_HD, 1)
    eye = jnp.eye(_HEADS, dtype=jnp.float32)
    return jnp.concatenate(
        [jnp.kron(eye, a_src), jnp.kron(eye, a_dst)], axis=1)


def kernel(node_feats, edge_index, W0, a0, g0, b0, W1, a1, g1, b1,
           W2, a2, g2, b2):
    n, d = node_feats.shape
    e = edge_index.shape[1]
    assert d == 128 and e % _NS == 0
    r = 1024
    npad = ((n + r - 1) // r) * r

    k = 400
    src = edge_index[0]
    dst = edge_index[1]
    z64 = jnp.zeros((npad // _NS, 64), jnp.float32)
    z8 = jnp.zeros((npad // _NS, 8), jnp.float32)
    p8 = jnp.concatenate(
        [jnp.kron(jnp.eye(_HEADS, dtype=jnp.float32),
                  jnp.ones((1, _HD), jnp.float32)),
         jnp.zeros((4, 128), jnp.float32)], axis=0)

    pre = _make_pre(npad, r)
    post = _make_post(npad, r)
    sc = _make_sc(npad, e, k)

    x = jnp.pad(node_feats, ((0, npad - n), (0, 0)))
    for (W, a, g, b) in ((W0, a0, g0, b0), (W1, a1, g1, b1), (W2, a2, g2, b2)):
        wx2, s = pre(x, W, _build_attn_mat(a))
        num, den = sc(wx2, s, src, dst, z64, z8)
        x = post(num, den, p8, g, b)
    return x[:n]


# speedup vs baseline: 84.5084x; 1.0797x over previous
import functools

import jax
import jax.numpy as jnp
from jax import lax
from jax.experimental import pallas as pl
from jax.experimental.pallas import tpu as pltpu
from jax.experimental.pallas import tpu_sc as plsc

_HEADS = 4
_HD = 32
_L = 16
_NC = 2
_NS = 16


def _pre_body(x_ref, w_ref, a_ref, wx2_ref, s_ref):
    wx = jnp.dot(x_ref[...], w_ref[...], preferred_element_type=jnp.float32)
    wx2_ref[0] = wx[:, :64]
    wx2_ref[1] = wx[:, 64:]
    s_ref[...] = jnp.dot(wx, a_ref[...], preferred_element_type=jnp.float32)


def _make_pre(n, r):
    return pl.pallas_call(
        _pre_body,
        grid=(n // r,),
        in_specs=[
            pl.BlockSpec((r, 128), lambda i: (i, 0)),
            pl.BlockSpec((128, 128), lambda i: (0, 0)),
            pl.BlockSpec((128, 8), lambda i: (0, 0)),
        ],
        out_specs=[
            pl.BlockSpec((_NC, r, 64), lambda i: (0, i, 0)),
            pl.BlockSpec((r, 8), lambda i: (i, 0)),
        ],
        out_shape=[
            jax.ShapeDtypeStruct((_NC, n, 64), jnp.float32),
            jax.ShapeDtypeStruct((n, 8), jnp.float32),
        ],
    )


def _post_body(num_ref, den_ref, p_ref, g_ref, b_ref, o_ref):
    num = jnp.concatenate([num_ref[0], num_ref[1]], axis=-1)
    den_exp = jnp.dot(den_ref[...], p_ref[...],
                      preferred_element_type=jnp.float32)
    h = num / (den_exp + 1e-8)
    h = jnp.where(h > 0, h, jnp.exp(h) - 1.0)
    mu = jnp.mean(h, axis=-1, keepdims=True)
    var = jnp.mean((h - mu) ** 2, axis=-1, keepdims=True)
    o_ref[...] = (h - mu) / jnp.sqrt(var + 1e-5) * g_ref[...] + b_ref[...]


def _make_post(n, r):
    return pl.pallas_call(
        _post_body,
        grid=(n // r,),
        in_specs=[
            pl.BlockSpec((_NC, r, 64), lambda i: (0, i, 0)),
            pl.BlockSpec((r, 8), lambda i: (i, 0)),
            pl.BlockSpec((8, 128), lambda i: (0, 0)),
            pl.BlockSpec((128,), lambda i: (0,)),
            pl.BlockSpec((128,), lambda i: (0,)),
        ],
        out_specs=pl.BlockSpec((r, 128), lambda i: (i, 0)),
        out_shape=jax.ShapeDtypeStruct((n, 128), jnp.float32),
    )


def _make_sc(n, e, k):
    assert k % _L == 0
    epw = e // _NS
    assert epw % k == 0
    ncl = epw // k
    assert ncl % 2 == 0
    rpt = n // _NS
    mesh = plsc.VectorSubcoreMesh(
        core_axis_name="c", subcore_axis_name="s",
        num_cores=_NC, num_subcores=_NS)

    @functools.partial(
        pl.kernel,
        out_type=[
            jax.ShapeDtypeStruct((_NC, n, 64), jnp.float32),
            jax.ShapeDtypeStruct((n, 8), jnp.float32),
        ],
        mesh=mesh,
        scratch_types=[
            pltpu.VMEM((k,), jnp.int32),
            pltpu.VMEM((k,), jnp.int32),
            pltpu.VMEM((k,), jnp.int32),
            pltpu.VMEM((k,), jnp.int32),
            pltpu.VMEM((k, 64), jnp.float32),
            pltpu.VMEM((k, 64), jnp.float32),
            pltpu.VMEM((k, 8), jnp.float32),
            pltpu.VMEM((k, 8), jnp.float32),
            pltpu.VMEM((k, 8), jnp.float32),
            pltpu.VMEM((k, 8), jnp.float32),
            pltpu.VMEM((k, 8), jnp.float32),
            pltpu.VMEM((k, 8), jnp.float32),
            pltpu.VMEM_SHARED((n, 64), jnp.float32),
            pltpu.VMEM_SHARED((n, 8), jnp.float32),
            pltpu.VMEM_SHARED((n, 8), jnp.float32),
            pltpu.SemaphoreType.DMA,
            pltpu.SemaphoreType.DMA,
            pltpu.SemaphoreType.DMA,
            pltpu.SemaphoreType.DMA,
            pltpu.SemaphoreType.DMA,
        ],
        compiler_params=pltpu.CompilerParams(
            needs_layout_passes=False, use_tc_tiling_on_sc=False),
    )
    def sc(wx2_hbm, s_hbm, src_hbm, dst_hbm, z64_hbm, z8_hbm,
           num_hbm, den_hbm,
           srcv_a, srcv_b, dstv_a, dstv_b,
           rows_a, rows_b, ssb_a, ssb_b, sdb_a, sdb_b, wbuf_a, wbuf_b,
           num_sh, den_sh, s_sh,
           isem_a, isem_b, rsem, ssem, wsem):
        cid = lax.axis_index("c")
        sid = lax.axis_index("s")
        nb = sid * rpt
        srcv = (srcv_a, srcv_b)
        dstv = (dstv_a, dstv_b)
        rows = (rows_a, rows_b)
        ssb = (ssb_a, ssb_b)
        sdb = (sdb_a, sdb_b)
        wbuf = (wbuf_a, wbuf_b)
        isem = (isem_a, isem_b)
        my_wx = wx2_hbm.at[cid]
        ebase = sid * epw

        pltpu.sync_copy(z64_hbm, num_sh.at[pl.ds(nb, rpt)])
        pltpu.sync_copy(z8_hbm, den_sh.at[pl.ds(nb, rpt)])
        pltpu.sync_copy(s_hbm.at[pl.ds(nb, rpt)], s_sh.at[pl.ds(nb, rpt)])
        pltpu.sync_copy(z8_hbm.at[pl.ds(0, k)], wbuf_a)
        pltpu.sync_copy(z8_hbm.at[pl.ds(0, k)], wbuf_b)
        plsc.subcore_barrier()

        def issue_idx(ci, p):
            cb = ebase + ci * k
            pltpu.async_copy(src_hbm.at[pl.ds(cb, k)], srcv[p], isem[p])
            pltpu.async_copy(dst_hbm.at[pl.ds(cb, k)], dstv[p], isem[p])

        def drain_idx(p):
            pltpu.make_async_copy(src_hbm.at[pl.ds(0, k)], srcv[p],
                                  isem[p]).wait()
            pltpu.make_async_copy(dst_hbm.at[pl.ds(0, k)], dstv[p],
                                  isem[p]).wait()

        def compute(p, d_rows, d_ssb, d_sdb):
            d_ssb.wait()
            d_sdb.wait()

            def wgrp(g, c2):
                idx = lax.iota(jnp.int32, _L) + g * _L
                for h in range(_HEADS):
                    hv = jnp.full((_L,), h, jnp.int32)
                    sa = plsc.load_gather(ssb[p], [idx, hv])
                    sb = plsc.load_gather(sdb[p], [idx, hv + 4])
                    ee = sa + sb
                    ee = jnp.maximum(ee, 0.2 * ee)
                    plsc.store_scatter(wbuf[p], [idx, hv], jnp.exp(ee))
                return c2
            lax.fori_loop(0, k // _L, wgrp, 0)

            d_rows.wait()
            rp = rows[p]

            def scale(i4, c2):
                for u in range(4):
                    i = i4 * 4 + u
                    iv = jnp.full((_L,), i, jnp.int32)
                    for h in range(2):
                        hv = jnp.full((_L,), h, jnp.int32) + 2 * cid
                        w = plsc.load_gather(wbuf[p], [iv, hv])
                        for half in range(2):
                            c0 = h * _HD + half * _L
                            rp[i, pl.ds(c0, _L)] = rp[i, pl.ds(c0, _L)] * w
                return c2
            lax.fori_loop(0, k // 4, scale, 0)

        issue_idx(0, 0)
        issue_idx(1, 1)

        def pair(t, carry):
            j0 = 2 * t
            drain_idx(0)
            drain_idx(1)
            d_r0 = pltpu.async_copy(my_wx.at[srcv[0]], rows[0], rsem)
            d_s0 = pltpu.async_copy(s_sh.at[srcv[0]], ssb[0], ssem)
            d_d0 = pltpu.async_copy(s_sh.at[dstv[0]], sdb[0], ssem)
            d_r1 = pltpu.async_copy(my_wx.at[srcv[1]], rows[1], rsem)
            d_s1 = pltpu.async_copy(s_sh.at[srcv[1]], ssb[1], ssem)
            d_d1 = pltpu.async_copy(s_sh.at[dstv[1]], sdb[1], ssem)

            compute(0, d_r0, d_s0, d_d0)
            dn0 = pltpu.async_copy(rows[0], num_sh.at[dstv[0]], wsem,
                                   add=True)
            dd0 = pltpu.async_copy(wbuf[0], den_sh.at[dstv[0]], wsem,
                                   add=True)
            compute(1, d_r1, d_s1, d_d1)
            dn0.wait()
            dd0.wait()

            @pl.when(j0 + 2 < ncl)
            def _():
                issue_idx(j0 + 2, 0)

            pltpu.sync_copy(rows[1], num_sh.at[dstv[1]], add=True)
            pltpu.sync_copy(wbuf[1], den_sh.at[dstv[1]], add=True)

            @pl.when(j0 + 3 < ncl)
            def _():
                issue_idx(j0 + 3, 1)
            return carry
        lax.fori_loop(0, ncl // 2, pair, 0)

        plsc.subcore_barrier()
        pltpu.sync_copy(num_sh.at[pl.ds(nb, rpt)],
                        num_hbm.at[cid, pl.ds(nb, rpt)])

        @pl.when(cid == 0)
        def _():
            pltpu.sync_copy(den_sh.at[pl.ds(nb, rpt)],
                            den_hbm.at[pl.ds(nb, rpt)])

    return sc


def _build_attn_mat(a):
    a_src = a[:_HD, 0].reshape(_HD, 1)
    a_dst = a[_HD:, 0].reshape(_HD, 1)
    eye = jnp.eye(_HEADS, dtype=jnp.float32)
    return jnp.concatenate(
        [jnp.kron(eye, a_src), jnp.kron(eye, a_dst)], axis=1)


def kernel(node_feats, edge_index, W0, a0, g0, b0, W1, a1, g1, b1,
           W2, a2, g2, b2):
    n, d = node_feats.shape
    e = edge_index.shape[1]
    assert d == 128 and e % _NS == 0
    r = 1024
    npad = ((n + r - 1) // r) * r

    k = 400
    src = edge_index[0]
    dst = edge_index[1]
    z64 = jnp.zeros((npad // _NS, 64), jnp.float32)
    z8 = jnp.zeros((npad // _NS, 8), jnp.float32)
    p8 = jnp.concatenate(
        [jnp.kron(jnp.eye(_HEADS, dtype=jnp.float32),
                  jnp.ones((1, _HD), jnp.float32)),
         jnp.zeros((4, 128), jnp.float32)], axis=0)

    pre = _make_pre(npad, r)
    post = _make_post(npad, r)
    sc = _make_sc(npad, e, k)

    x = jnp.pad(node_feats, ((0, npad - n), (0, 0)))
    for (W, a, g, b) in ((W0, a0, g0, b0), (W1, a1, g1, b1), (W2, a2, g2, b2)):
        wx2, s = pre(x, W, _build_attn_mat(a))
        num, den = sc(wx2, s, src, dst, z64, z8)
        x = post(num, den, p8, g, b)
    return x[:n]


# speedup vs baseline: 85.7905x; 1.0152x over previous
import functools

import jax
import jax.numpy as jnp
from jax import lax
from jax.experimental import pallas as pl
from jax.experimental.pallas import tpu as pltpu
from jax.experimental.pallas import tpu_sc as plsc

_HEADS = 4
_HD = 32
_L = 16
_NC = 2
_NS = 16


def _pre_body(x_ref, w_ref, a_ref, wx2_ref, s_ref):
    wx = jnp.dot(x_ref[...], w_ref[...], preferred_element_type=jnp.float32)
    wx2_ref[0] = wx[:, :64]
    wx2_ref[1] = wx[:, 64:]
    s_ref[...] = jnp.dot(wx, a_ref[...], preferred_element_type=jnp.float32)


def _make_pre(n, r):
    return pl.pallas_call(
        _pre_body,
        grid=(n // r,),
        in_specs=[
            pl.BlockSpec((r, 128), lambda i: (i, 0)),
            pl.BlockSpec((128, 128), lambda i: (0, 0)),
            pl.BlockSpec((128, 8), lambda i: (0, 0)),
        ],
        out_specs=[
            pl.BlockSpec((_NC, r, 64), lambda i: (0, i, 0)),
            pl.BlockSpec((r, 8), lambda i: (i, 0)),
        ],
        out_shape=[
            jax.ShapeDtypeStruct((_NC, n, 64), jnp.float32),
            jax.ShapeDtypeStruct((n, 8), jnp.float32),
        ],
    )


def _post_body(num_ref, den_ref, p_ref, g_ref, b_ref, o_ref):
    num = jnp.concatenate([num_ref[0], num_ref[1]], axis=-1)
    den_exp = jnp.dot(den_ref[...], p_ref[...],
                      preferred_element_type=jnp.float32)
    h = num / (den_exp + 1e-8)
    h = jnp.where(h > 0, h, jnp.exp(h) - 1.0)
    mu = jnp.mean(h, axis=-1, keepdims=True)
    var = jnp.mean((h - mu) ** 2, axis=-1, keepdims=True)
    o_ref[...] = (h - mu) / jnp.sqrt(var + 1e-5) * g_ref[...] + b_ref[...]


def _make_post(n, r):
    return pl.pallas_call(
        _post_body,
        grid=(n // r,),
        in_specs=[
            pl.BlockSpec((_NC, r, 64), lambda i: (0, i, 0)),
            pl.BlockSpec((r, 8), lambda i: (i, 0)),
            pl.BlockSpec((8, 128), lambda i: (0, 0)),
            pl.BlockSpec((128,), lambda i: (0,)),
            pl.BlockSpec((128,), lambda i: (0,)),
        ],
        out_specs=pl.BlockSpec((r, 128), lambda i: (i, 0)),
        out_shape=jax.ShapeDtypeStruct((n, 128), jnp.float32),
    )


def _make_sc(n, e, k):
    assert k % _L == 0
    epw = e // _NS
    assert epw % k == 0
    ncl = epw // k
    assert ncl % 2 == 0
    rpt = n // _NS
    mesh = plsc.VectorSubcoreMesh(
        core_axis_name="c", subcore_axis_name="s",
        num_cores=_NC, num_subcores=_NS)

    @functools.partial(
        pl.kernel,
        out_type=[
            jax.ShapeDtypeStruct((_NC, n, 64), jnp.float32),
            jax.ShapeDtypeStruct((n, 8), jnp.float32),
        ],
        mesh=mesh,
        scratch_types=[
            pltpu.VMEM((k,), jnp.int32),
            pltpu.VMEM((k,), jnp.int32),
            pltpu.VMEM((k,), jnp.int32),
            pltpu.VMEM((k,), jnp.int32),
            pltpu.VMEM((k, 64), jnp.float32),
            pltpu.VMEM((k, 64), jnp.float32),
            pltpu.VMEM((k, 8), jnp.float32),
            pltpu.VMEM((k, 8), jnp.float32),
            pltpu.VMEM((k, 8), jnp.float32),
            pltpu.VMEM((k, 8), jnp.float32),
            pltpu.VMEM((k, 8), jnp.float32),
            pltpu.VMEM((k, 8), jnp.float32),
            pltpu.VMEM_SHARED((n, 64), jnp.float32),
            pltpu.VMEM_SHARED((n, 8), jnp.float32),
            pltpu.VMEM_SHARED((n, 8), jnp.float32),
            pltpu.SemaphoreType.DMA,
            pltpu.SemaphoreType.DMA,
            pltpu.SemaphoreType.DMA,
            pltpu.SemaphoreType.DMA,
            pltpu.SemaphoreType.DMA,
        ],
        compiler_params=pltpu.CompilerParams(
            needs_layout_passes=False, use_tc_tiling_on_sc=False),
    )
    def sc(wx2_hbm, s_hbm, src_hbm, dst_hbm, z64_hbm, z8_hbm,
           num_hbm, den_hbm,
           srcv_a, srcv_b, dstv_a, dstv_b,
           rows_a, rows_b, ssb_a, ssb_b, sdb_a, sdb_b, wbuf_a, wbuf_b,
           num_sh, den_sh, s_sh,
           isem_a, isem_b, rsem, ssem, wsem):
        cid = lax.axis_index("c")
        sid = lax.axis_index("s")
        nb = sid * rpt
        srcv = (srcv_a, srcv_b)
        dstv = (dstv_a, dstv_b)
        rows = (rows_a, rows_b)
        ssb = (ssb_a, ssb_b)
        sdb = (sdb_a, sdb_b)
        wbuf = (wbuf_a, wbuf_b)
        isem = (isem_a, isem_b)
        my_wx = wx2_hbm.at[cid]
        ebase = sid * epw

        pltpu.sync_copy(z64_hbm, num_sh.at[pl.ds(nb, rpt)])
        pltpu.sync_copy(z8_hbm, den_sh.at[pl.ds(nb, rpt)])
        pltpu.sync_copy(s_hbm.at[pl.ds(nb, rpt)], s_sh.at[pl.ds(nb, rpt)])
        pltpu.sync_copy(z8_hbm.at[pl.ds(0, k)], wbuf_a)
        pltpu.sync_copy(z8_hbm.at[pl.ds(0, k)], wbuf_b)
        plsc.subcore_barrier()

        def issue_idx(ci, p):
            cb = ebase + ci * k
            pltpu.async_copy(src_hbm.at[pl.ds(cb, k)], srcv[p], isem[p])
            pltpu.async_copy(dst_hbm.at[pl.ds(cb, k)], dstv[p], isem[p])

        def drain_idx(p):
            pltpu.make_async_copy(src_hbm.at[pl.ds(0, k)], srcv[p],
                                  isem[p]).wait()
            pltpu.make_async_copy(dst_hbm.at[pl.ds(0, k)], dstv[p],
                                  isem[p]).wait()

        def wcompute(p, d_ssb, d_sdb):
            d_ssb.wait()
            d_sdb.wait()

            def wgrp(g, c2):
                idx = lax.iota(jnp.int32, _L) + g * _L
                for h in range(_HEADS):
                    hv = jnp.full((_L,), h, jnp.int32)
                    sa = plsc.load_gather(ssb[p], [idx, hv])
                    sb = plsc.load_gather(sdb[p], [idx, hv + 4])
                    ee = sa + sb
                    ee = jnp.maximum(ee, 0.2 * ee)
                    plsc.store_scatter(wbuf[p], [idx, hv], jnp.exp(ee))
                return c2
            lax.fori_loop(0, k // _L, wgrp, 0)

        def scalerows(p, d_rows):
            d_rows.wait()
            rp = rows[p]

            def scale(i8, c2):
                for u in range(8):
                    i = i8 * 8 + u
                    iv = jnp.full((_L,), i, jnp.int32)
                    for h in range(2):
                        hv = jnp.full((_L,), h, jnp.int32) + 2 * cid
                        w = plsc.load_gather(wbuf[p], [iv, hv])
                        for half in range(2):
                            c0 = h * _HD + half * _L
                            rp[i, pl.ds(c0, _L)] = rp[i, pl.ds(c0, _L)] * w
                return c2
            lax.fori_loop(0, k // 8, scale, 0)

        issue_idx(0, 0)
        issue_idx(1, 1)

        def pair(t, carry):
            j0 = 2 * t
            drain_idx(0)
            drain_idx(1)
            d_r0 = pltpu.async_copy(my_wx.at[srcv[0]], rows[0], rsem)
            d_s0 = pltpu.async_copy(s_sh.at[srcv[0]], ssb[0], ssem)
            d_d0 = pltpu.async_copy(s_sh.at[dstv[0]], sdb[0], ssem)
            d_r1 = pltpu.async_copy(my_wx.at[srcv[1]], rows[1], rsem)
            d_s1 = pltpu.async_copy(s_sh.at[srcv[1]], ssb[1], ssem)
            d_d1 = pltpu.async_copy(s_sh.at[dstv[1]], sdb[1], ssem)

            wcompute(0, d_s0, d_d0)
            wcompute(1, d_s1, d_d1)
            scalerows(0, d_r0)
            dn0 = pltpu.async_copy(rows[0], num_sh.at[dstv[0]], wsem,
                                   add=True)
            dd0 = pltpu.async_copy(wbuf[0], den_sh.at[dstv[0]], wsem,
                                   add=True)
            scalerows(1, d_r1)
            dn0.wait()
            dd0.wait()

            @pl.when(j0 + 2 < ncl)
            def _():
                issue_idx(j0 + 2, 0)

            pltpu.sync_copy(rows[1], num_sh.at[dstv[1]], add=True)
            pltpu.sync_copy(wbuf[1], den_sh.at[dstv[1]], add=True)

            @pl.when(j0 + 3 < ncl)
            def _():
                issue_idx(j0 + 3, 1)
            return carry
        lax.fori_loop(0, ncl // 2, pair, 0)

        plsc.subcore_barrier()
        pltpu.sync_copy(num_sh.at[pl.ds(nb, rpt)],
                        num_hbm.at[cid, pl.ds(nb, rpt)])

        @pl.when(cid == 0)
        def _():
            pltpu.sync_copy(den_sh.at[pl.ds(nb, rpt)],
                            den_hbm.at[pl.ds(nb, rpt)])

    return sc


def _build_attn_mat(a):
    a_src = a[:_HD, 0].reshape(_HD, 1)
    a_dst = a[_HD:, 0].reshape(_HD, 1)
    eye = jnp.eye(_HEADS, dtype=jnp.float32)
    return jnp.concatenate(
        [jnp.kron(eye, a_src), jnp.kron(eye, a_dst)], axis=1)


def kernel(node_feats, edge_index, W0, a0, g0, b0, W1, a1, g1, b1,
           W2, a2, g2, b2):
    n, d = node_feats.shape
    e = edge_index.shape[1]
    assert d == 128 and e % _NS == 0
    r = 1024
    npad = ((n + r - 1) // r) * r

    k = 400
    src = edge_index[0]
    dst = edge_index[1]
    z64 = jnp.zeros((npad // _NS, 64), jnp.float32)
    z8 = jnp.zeros((npad // _NS, 8), jnp.float32)
    p8 = jnp.concatenate(
        [jnp.kron(jnp.eye(_HEADS, dtype=jnp.float32),
                  jnp.ones((1, _HD), jnp.float32)),
         jnp.zeros((4, 128), jnp.float32)], axis=0)

    pre = _make_pre(npad, r)
    post = _make_post(npad, r)
    sc = _make_sc(npad, e, k)

    x = jnp.pad(node_feats, ((0, npad - n), (0, 0)))
    for (W, a, g, b) in ((W0, a0, g0, b0), (W1, a1, g1, b1), (W2, a2, g2, b2)):
        wx2, s = pre(x, W, _build_attn_mat(a))
        num, den = sc(wx2, s, src, dst, z64, z8)
        x = post(num, den, p8, g, b)
    return x[:n]
